# baseline (speedup 1.0000x reference)
import functools

import jax
import jax.numpy as jnp
from jax import lax
from jax.experimental import pallas as pl
from jax.experimental.pallas import tpu as pltpu
from jax.experimental.pallas import tpu_sc as plsc

N_EMBD = 64
N_TAB = 102048
B = 1024
L = 200

BLO = 128
BHI = B // BLO
EHI = N_EMBD // 8

_info = plsc.get_sparse_core_info()
NC = _info.num_cores
NS = _info.num_subcores
NW = NC * NS

LGROUPS = NW // BHI
L_PER_W = L // LGROUPS


def _tec_body(idx_hbm, tab_hbm, out_hbm, idx_v, rows_a, rows_b, out_v, gsem, osem):
    wid = lax.axis_index("s") * NC + lax.axis_index("c")
    j = wid // BHI
    h = wid % BHI

    pltpu.sync_copy(idx_hbm.at[pl.ds(j * L_PER_W, L_PER_W), h], idx_v)

    iota16 = lax.iota(jnp.int32, 16)

    def fire(k, buf):
        pltpu.async_copy(tab_hbm.at[idx_v.at[k, 0]], rows_a.at[buf], gsem)
        pltpu.async_copy(tab_hbm.at[idx_v.at[k, 1]], rows_b.at[buf], gsem)

    fire(0, 0)

    def do_chunk(k, buf):
        @pl.when(k + 1 < L_PER_W)
        def _():
            fire(k + 1, 1 - buf)

        pltpu.make_async_copy(tab_hbm.at[idx_v.at[k, 0]], rows_a.at[buf], gsem).wait()
        pltpu.make_async_copy(tab_hbm.at[idx_v.at[k, 1]], rows_b.at[buf], gsem).wait()

        @pl.when(k >= 2)
        def _():
            for e in range(EHI):
                pltpu.make_async_copy(
                    out_v.at[buf, pl.ds(8 * e, 8)], out_hbm.at[0, 0, 0], osem
                ).wait()

        def col(e, carry):
            esplat = jnp.zeros((16,), jnp.int32) + e
            for bq in range(BLO // 16):
                bidx = iota16 + 16 * bq
                va = plsc.load_gather(rows_a.at[buf], [bidx, esplat])
                vb = plsc.load_gather(rows_b.at[buf], [bidx, esplat])
                out_v[buf, e, pl.ds(16 * bq, 16)] = va + vb
            return carry

        lax.fori_loop(0, N_EMBD, col, 0, unroll=2)

        for e in range(EHI):
            pltpu.async_copy(
                out_v.at[buf, pl.ds(8 * e, 8)],
                out_hbm.at[j * L_PER_W + k, e, h],
                osem,
            )

    def outer(i, carry):
        do_chunk(2 * i, 0)
        do_chunk(2 * i + 1, 1)
        return carry

    lax.fori_loop(0, L_PER_W // 2, outer, 0)

    for buf in range(2):
        for e in range(EHI):
            pltpu.make_async_copy(
                out_v.at[buf, pl.ds(8 * e, 8)], out_hbm.at[0, 0, 0], osem
            ).wait()


@functools.partial(
    pl.kernel,
    mesh=plsc.VectorSubcoreMesh(core_axis_name="c", subcore_axis_name="s"),
    compiler_params=pltpu.CompilerParams(
        use_tc_tiling_on_sc=False, needs_layout_passes=False
    ),
    out_type=jax.ShapeDtypeStruct((L, EHI, BHI, 8, BLO), jnp.float32),
    scratch_types=[
        pltpu.VMEM((L_PER_W, 2, BLO), jnp.int32),
        pltpu.VMEM((2, BLO, N_EMBD), jnp.float32),
        pltpu.VMEM((2, BLO, N_EMBD), jnp.float32),
        pltpu.VMEM((2, N_EMBD, BLO), jnp.float32),
        pltpu.SemaphoreType.DMA,
        pltpu.SemaphoreType.DMA,
    ],
)
def _embed_sum(idx_hbm, tab_hbm, out_hbm, idx_v, rows_a, rows_b, out_v, gsem, osem):
    _tec_body(idx_hbm, tab_hbm, out_hbm, idx_v, rows_a, rows_b, out_v, gsem, osem)


@jax.jit
def kernel(inputs, we):
    idx = inputs.astype(jnp.int32).reshape(BHI, BLO, L, 2).transpose(2, 0, 3, 1)
    out = _embed_sum(idx, we)
    return out.transpose(2, 4, 0, 1, 3).reshape(B, L, N_EMBD)


# speedup vs baseline: 2.9026x; 2.9026x over previous
import functools

import jax
import jax.numpy as jnp
from jax import lax
from jax.experimental import pallas as pl
from jax.experimental.pallas import tpu as pltpu
from jax.experimental.pallas import tpu_sc as plsc

N_EMBD = 64
N_TAB = 102048
B = 1024
L = 200

BLO = 128
BHI = B // BLO
EHI = N_EMBD // 8
OPAD = BLO + 1

_info = plsc.get_sparse_core_info()
NC = _info.num_cores
NS = _info.num_subcores
NW = NC * NS

LGROUPS = NW // BHI
L_PER_W = L // LGROUPS


def _tec_body(idx_hbm, tab_hbm, out_hbm, idx_v, rows_a, rows_b, out_v, gsem, osem):
    wid = lax.axis_index("s") * NC + lax.axis_index("c")
    j = wid // BHI
    h = wid % BHI

    pltpu.sync_copy(idx_hbm.at[pl.ds(j * L_PER_W, L_PER_W), h], idx_v)

    iota16 = lax.iota(jnp.int32, 16)

    def fire(k, buf):
        pltpu.async_copy(tab_hbm.at[idx_v.at[k, 0]], rows_a.at[buf], gsem)
        pltpu.async_copy(tab_hbm.at[idx_v.at[k, 1]], rows_b.at[buf], gsem)

    fire(0, 0)

    def do_chunk(k, buf):
        @pl.when(k + 1 < L_PER_W)
        def _():
            fire(k + 1, 1 - buf)

        pltpu.make_async_copy(tab_hbm.at[idx_v.at[k, 0]], rows_a.at[buf], gsem).wait()
        pltpu.make_async_copy(tab_hbm.at[idx_v.at[k, 1]], rows_b.at[buf], gsem).wait()

        @pl.when(k >= 2)
        def _():
            for e in range(EHI):
                pltpu.make_async_copy(
                    out_v.at[buf, pl.ds(8 * e, 8), pl.ds(0, BLO)],
                    out_hbm.at[0, 0, 0],
                    osem,
                ).wait()

        def row(m, carry):
            msplat = jnp.zeros((16,), jnp.int32) + m
            for eq in range(N_EMBD // 16):
                v = (
                    rows_a[buf, m, pl.ds(16 * eq, 16)]
                    + rows_b[buf, m, pl.ds(16 * eq, 16)]
                )
                plsc.store_scatter(
                    out_v.at[buf], [iota16 + 16 * eq, msplat], v
                )
            return carry

        lax.fori_loop(0, BLO, row, 0, unroll=2)

        for e in range(EHI):
            pltpu.async_copy(
                out_v.at[buf, pl.ds(8 * e, 8), pl.ds(0, BLO)],
                out_hbm.at[j * L_PER_W + k, e, h],
                osem,
            )

    def outer(i, carry):
        do_chunk(2 * i, 0)
        do_chunk(2 * i + 1, 1)
        return carry

    lax.fori_loop(0, L_PER_W // 2, outer, 0)

    for buf in range(2):
        for e in range(EHI):
            pltpu.make_async_copy(
                out_v.at[buf, pl.ds(8 * e, 8), pl.ds(0, BLO)],
                out_hbm.at[0, 0, 0],
                osem,
            ).wait()


@functools.partial(
    pl.kernel,
    mesh=plsc.VectorSubcoreMesh(core_axis_name="c", subcore_axis_name="s"),
    compiler_params=pltpu.CompilerParams(
        use_tc_tiling_on_sc=False, needs_layout_passes=False
    ),
    out_type=jax.ShapeDtypeStruct((L, EHI, BHI, 8, BLO), jnp.float32),
    scratch_types=[
        pltpu.VMEM((L_PER_W, 2, BLO), jnp.int32),
        pltpu.VMEM((2, BLO, N_EMBD), jnp.float32),
        pltpu.VMEM((2, BLO, N_EMBD), jnp.float32),
        pltpu.VMEM((2, N_EMBD, OPAD), jnp.float32),
        pltpu.SemaphoreType.DMA,
        pltpu.SemaphoreType.DMA,
    ],
)
def _embed_sum(idx_hbm, tab_hbm, out_hbm, idx_v, rows_a, rows_b, out_v, gsem, osem):
    _tec_body(idx_hbm, tab_hbm, out_hbm, idx_v, rows_a, rows_b, out_v, gsem, osem)


@jax.jit
def kernel(inputs, we):
    idx = inputs.astype(jnp.int32).reshape(BHI, BLO, L, 2).transpose(2, 0, 3, 1)
    out = _embed_sum(idx, we)
    return out.transpose(2, 4, 0, 1, 3).reshape(B, L, N_EMBD)
